# baseline (speedup 1.0000x reference)
import jax
import jax.numpy as jnp
from jax import lax
from jax.experimental import pallas as pl
from jax.experimental.pallas import tpu as pltpu

B, S, P = 2, 128, 256
D = 1024
H = 16
DH = D // H

T_D = 2
DT = D // T_D
T_O = 4
DO = D // T_O

_XWT = (((1,), (1,)), ((), ()))


def _attn_body(shelf_ref, product_ref, supply_ref, wq_ref, bq_ref,
               wk_ref, bk_ref, wv_ref, bv_ref, attn_ref,
               q_acc, k_acc, v_acc):
    t = pl.program_id(0)
    b = pl.program_id(1)

    x_s = shelf_ref[0]
    x_p = product_ref[0]

    dq = lax.dot_general(x_s, wq_ref[...], _XWT, preferred_element_type=jnp.float32)
    dk = lax.dot_general(x_p, wk_ref[...], _XWT, preferred_element_type=jnp.float32)
    dv = lax.dot_general(x_p, wv_ref[...], _XWT, preferred_element_type=jnp.float32)

    @pl.when(t == 0)
    def _init():
        q_acc[b] = dq + bq_ref[...]
        k_acc[b] = dk + bk_ref[...]
        v_acc[b] = dv + bv_ref[...]

    @pl.when(t != 0)
    def _accum():
        q_acc[b] += dq
        k_acc[b] += dk
        v_acc[b] += dv

    @pl.when(t == T_D - 1)
    def _attend():
        q4 = q_acc[b].reshape(S, H, DH)
        k4 = k_acc[b].reshape(P, H, DH)
        v4 = v_acc[b].reshape(P, H, DH)

        logits = lax.dot_general(
            q4, k4,
            dimension_numbers=(((2,), (2,)), ((1,), (1,))),
            preferred_element_type=jnp.float32,
        ) * (1.0 / (DH ** 0.5))

        mask = (supply_ref[0] > 0)[None, :, :]
        masked = jnp.where(mask, logits, -1e30)
        m = jnp.max(masked, axis=2, keepdims=True)
        e = jnp.where(mask, jnp.exp(logits - m), 0.0)
        den = jnp.sum(e, axis=2, keepdims=True)
        w = e / (den + 1e-9)

        attn_ref[0] = lax.dot_general(
            w, v4,
            dimension_numbers=(((2,), (0,)), ((0,), (1,))),
            preferred_element_type=jnp.float32,
        )


def _oproj_body(x_ref, wo_ref, bo_ref, out_ref):
    out_ref[...] = lax.dot_general(x_ref[...], wo_ref[...], _XWT,
                                   preferred_element_type=jnp.float32) + bo_ref[...]


@jax.jit
def kernel(shelf_embs, product_embs, supply, W_q, b_q, W_k, b_k, W_v, b_v, W_o, b_o):
    attn = pl.pallas_call(
        _attn_body,
        grid=(T_D, B),
        in_specs=[
            pl.BlockSpec((1, S, DT), lambda t, b: (b, 0, t)),
            pl.BlockSpec((1, P, DT), lambda t, b: (b, 0, t)),
            pl.BlockSpec((1, S, P), lambda t, b: (b, 0, 0)),
            pl.BlockSpec((D, DT), lambda t, b: (0, t)),
            pl.BlockSpec((D,), lambda t, b: (0,)),
            pl.BlockSpec((D, DT), lambda t, b: (0, t)),
            pl.BlockSpec((D,), lambda t, b: (0,)),
            pl.BlockSpec((D, DT), lambda t, b: (0, t)),
            pl.BlockSpec((D,), lambda t, b: (0,)),
        ],
        out_specs=pl.BlockSpec((1, H, S, DH), lambda t, b: (b, 0, 0, 0)),
        out_shape=jax.ShapeDtypeStruct((B, H, S, DH), jnp.float32),
        scratch_shapes=[
            pltpu.VMEM((B, S, D), jnp.float32),
            pltpu.VMEM((B, P, D), jnp.float32),
            pltpu.VMEM((B, P, D), jnp.float32),
        ],
    )(shelf_embs, product_embs, supply, W_q, b_q, W_k, b_k, W_v, b_v)

    scr = attn.reshape(B * S, D)

    out = pl.pallas_call(
        _oproj_body,
        grid=(T_O,),
        in_specs=[
            pl.BlockSpec((B * S, D), lambda t: (0, 0)),
            pl.BlockSpec((DO, D), lambda t: (t, 0)),
            pl.BlockSpec((DO,), lambda t: (t,)),
        ],
        out_specs=pl.BlockSpec((B * S, DO), lambda t: (0, t)),
        out_shape=jax.ShapeDtypeStruct((B * S, D), jnp.float32),
    )(scr, W_o, b_o)
    return out.reshape(B, S, D)


# speedup vs baseline: 1.3684x; 1.3684x over previous
import jax
import jax.numpy as jnp
from jax import lax
from jax.experimental import pallas as pl

B, S, P = 2, 128, 256
D = 1024
H = 16
DH = D // H

G = 4
HPG = H // G
DG = D // G

_XWT = (((1,), (1,)), ((), ()))


def _attn_body(shelf_ref, product_ref, supply_ref, wq_ref, bq_ref,
               wk_ref, bk_ref, wv_ref, bv_ref, attn_ref):
    b = pl.program_id(1)

    x_s = shelf_ref[b]
    x_p = product_ref[b]

    q = lax.dot_general(x_s, wq_ref[...], _XWT,
                        preferred_element_type=jnp.float32) + bq_ref[...]
    k = lax.dot_general(x_p, wk_ref[...], _XWT,
                        preferred_element_type=jnp.float32) + bk_ref[...]
    v = lax.dot_general(x_p, wv_ref[...], _XWT,
                        preferred_element_type=jnp.float32) + bv_ref[...]

    q4 = q.reshape(S, HPG, DH)
    k4 = k.reshape(P, HPG, DH)
    v4 = v.reshape(P, HPG, DH)

    logits = lax.dot_general(
        q4, k4,
        dimension_numbers=(((2,), (2,)), ((1,), (1,))),
        preferred_element_type=jnp.float32,
    ) * (1.0 / (DH ** 0.5))

    mask = (supply_ref[b] > 0)[None, :, :]
    masked = jnp.where(mask, logits, -1e30)
    m = jnp.max(masked, axis=2, keepdims=True)
    e = jnp.where(mask, jnp.exp(logits - m), 0.0)
    den = jnp.sum(e, axis=2, keepdims=True)
    w = e / (den + 1e-9)

    attn_ref[0] = lax.dot_general(
        w, v4,
        dimension_numbers=(((2,), (0,)), ((0,), (1,))),
        preferred_element_type=jnp.float32,
    )


def _oproj_body(x_ref, wo_ref, bo_ref, out_ref):
    out_ref[...] = lax.dot_general(x_ref[...], wo_ref[...], _XWT,
                                   preferred_element_type=jnp.float32) + bo_ref[...]


T_O = 4
DO = D // T_O


@jax.jit
def kernel(shelf_embs, product_embs, supply, W_q, b_q, W_k, b_k, W_v, b_v, W_o, b_o):
    attn = pl.pallas_call(
        _attn_body,
        grid=(G, B),
        in_specs=[
            pl.BlockSpec((B, S, D), lambda g, b: (0, 0, 0)),
            pl.BlockSpec((B, P, D), lambda g, b: (0, 0, 0)),
            pl.BlockSpec((B, S, P), lambda g, b: (0, 0, 0)),
            pl.BlockSpec((DG, D), lambda g, b: (g, 0)),
            pl.BlockSpec((DG,), lambda g, b: (g,)),
            pl.BlockSpec((DG, D), lambda g, b: (g, 0)),
            pl.BlockSpec((DG,), lambda g, b: (g,)),
            pl.BlockSpec((DG, D), lambda g, b: (g, 0)),
            pl.BlockSpec((DG,), lambda g, b: (g,)),
        ],
        out_specs=pl.BlockSpec((1, HPG, S, DH), lambda g, b: (b, g, 0, 0)),
        out_shape=jax.ShapeDtypeStruct((B, H, S, DH), jnp.float32),
    )(shelf_embs, product_embs, supply, W_q, b_q, W_k, b_k, W_v, b_v)

    scr = attn.reshape(B * S, D)

    out = pl.pallas_call(
        _oproj_body,
        grid=(T_O,),
        in_specs=[
            pl.BlockSpec((B * S, D), lambda t: (0, 0)),
            pl.BlockSpec((DO, D), lambda t: (t, 0)),
            pl.BlockSpec((DO,), lambda t: (t,)),
        ],
        out_specs=pl.BlockSpec((B * S, DO), lambda t: (0, t)),
        out_shape=jax.ShapeDtypeStruct((B * S, D), jnp.float32),
    )(scr, W_o, b_o)
    return out.reshape(B, S, D)


# speedup vs baseline: 1.8280x; 1.3359x over previous
import jax
import jax.numpy as jnp
from jax import lax
from jax.experimental import pallas as pl
from jax.experimental.pallas import tpu as pltpu

B, S, P = 2, 128, 256
D = 1024
H = 16
DH = D // H

_XWT = (((1,), (1,)), ((), ()))


def _fused_body(shelf_ref, product_ref, supply_ref, wq_ref, bq_ref,
                wk_ref, bk_ref, wv_ref, bv_ref, wo_ref, bo_ref,
                out_ref, wo3_s):
    @pl.when(pl.program_id(0) == 0)
    def _prep():
        wo3_s[...] = wo_ref[...].T.reshape(16, DH, D)

    x_s = shelf_ref[0]
    x_p = product_ref[0]

    q = lax.dot_general(x_s, wq_ref[...], _XWT,
                        preferred_element_type=jnp.float32) + bq_ref[...]
    k = lax.dot_general(x_p, wk_ref[...], _XWT,
                        preferred_element_type=jnp.float32) + bk_ref[...]
    v = lax.dot_general(x_p, wv_ref[...], _XWT,
                        preferred_element_type=jnp.float32) + bv_ref[...]

    q4 = q.reshape(S, H, DH)
    k4 = k.reshape(P, H, DH)
    v4 = v.reshape(P, H, DH)

    logits = lax.dot_general(
        q4, k4,
        dimension_numbers=(((2,), (2,)), ((1,), (1,))),
        preferred_element_type=jnp.float32,
    ) * (1.0 / (DH ** 0.5))

    mask = (supply_ref[0] > 0)[None, :, :]
    masked = jnp.where(mask, logits, -1e30)
    m = jnp.max(masked, axis=2, keepdims=True)
    e = jnp.where(mask, jnp.exp(logits - m), 0.0)
    den = jnp.sum(e, axis=2, keepdims=True)
    w = e / (den + 1e-9)

    attn = lax.dot_general(
        w, v4,
        dimension_numbers=(((2,), (0,)), ((0,), (1,))),
        preferred_element_type=jnp.float32,
    )

    a4 = attn.reshape(H, 8, 16, DH)
    t = lax.dot_general(
        a4, wo3_s[...],
        dimension_numbers=(((3,), (1,)), ((2,), (0,))),
        preferred_element_type=jnp.float32,
    )
    out_ref[0] = jnp.sum(t, axis=0).reshape(S, D) + bo_ref[...]


@jax.jit
def kernel(shelf_embs, product_embs, supply, W_q, b_q, W_k, b_k, W_v, b_v, W_o, b_o):
    out = pl.pallas_call(
        _fused_body,
        grid=(B,),
        in_specs=[
            pl.BlockSpec((1, S, D), lambda b: (b, 0, 0)),
            pl.BlockSpec((1, P, D), lambda b: (b, 0, 0)),
            pl.BlockSpec((1, S, P), lambda b: (b, 0, 0)),
            pl.BlockSpec((D, D), lambda b: (0, 0)),
            pl.BlockSpec((D,), lambda b: (0,)),
            pl.BlockSpec((D, D), lambda b: (0, 0)),
            pl.BlockSpec((D,), lambda b: (0,)),
            pl.BlockSpec((D, D), lambda b: (0, 0)),
            pl.BlockSpec((D,), lambda b: (0,)),
            pl.BlockSpec((D, D), lambda b: (0, 0)),
            pl.BlockSpec((D,), lambda b: (0,)),
        ],
        out_specs=pl.BlockSpec((1, S, D), lambda b: (b, 0, 0)),
        out_shape=jax.ShapeDtypeStruct((B, S, D), jnp.float32),
        scratch_shapes=[pltpu.VMEM((16, DH, D), jnp.float32)],
    )(shelf_embs, product_embs, supply, W_q, b_q, W_k, b_k, W_v, b_v, W_o, b_o)
    return out


# speedup vs baseline: 1.8474x; 1.0106x over previous
import jax
import jax.numpy as jnp
from jax import lax
from jax.experimental import pallas as pl
from jax.experimental.pallas import tpu as pltpu

B, S, P = 2, 128, 256
D = 1024
H = 16
DH = D // H

_XWT = (((1,), (1,)), ((), ()))


def _fused_body(shelf_ref, product_ref, supply_ref, wq_ref, bq_ref,
                wk_ref, bk_ref, wv_ref, bv_ref, wo_ref, bo_ref,
                out_ref, wo3_s):
    @pl.when(pl.program_id(0) == 0)
    def _prep():
        wo3_s[...] = wo_ref[...].T.reshape(16, DH, D)

    x_s = shelf_ref[0]
    x_p = product_ref[0]

    q = lax.dot_general(x_s, wq_ref[...], _XWT,
                        preferred_element_type=jnp.float32) + bq_ref[...]
    k = lax.dot_general(x_p, wk_ref[...], _XWT,
                        preferred_element_type=jnp.float32) + bk_ref[...]
    v = lax.dot_general(x_p, wv_ref[...], _XWT,
                        preferred_element_type=jnp.float32) + bv_ref[...]

    q4 = q.reshape(S, H, DH)
    k4 = k.reshape(P, H, DH)
    v4 = v.reshape(P, H, DH)

    logits = lax.dot_general(
        q4, k4,
        dimension_numbers=(((2,), (2,)), ((1,), (1,))),
        preferred_element_type=jnp.float32,
    ) * (1.0 / (DH ** 0.5))

    mask = (supply_ref[0] > 0)[None, :, :]
    m = jnp.max(logits, axis=2, keepdims=True)
    e = jnp.where(mask, jnp.exp(logits - m), 0.0)
    den = jnp.sum(e, axis=2, keepdims=True)
    w = e / (den + 1e-9)

    attn = lax.dot_general(
        w, v4,
        dimension_numbers=(((2,), (0,)), ((0,), (1,))),
        preferred_element_type=jnp.float32,
    )

    a4 = attn.reshape(H, 8, 16, DH)
    t = lax.dot_general(
        a4, wo3_s[...],
        dimension_numbers=(((3,), (1,)), ((2,), (0,))),
        preferred_element_type=jnp.float32,
    )
    out_ref[0] = jnp.sum(t, axis=0).reshape(S, D) + bo_ref[...]


@jax.jit
def kernel(shelf_embs, product_embs, supply, W_q, b_q, W_k, b_k, W_v, b_v, W_o, b_o):
    out = pl.pallas_call(
        _fused_body,
        grid=(B,),
        in_specs=[
            pl.BlockSpec((1, S, D), lambda b: (b, 0, 0)),
            pl.BlockSpec((1, P, D), lambda b: (b, 0, 0)),
            pl.BlockSpec((1, S, P), lambda b: (b, 0, 0)),
            pl.BlockSpec((D, D), lambda b: (0, 0)),
            pl.BlockSpec((D,), lambda b: (0,)),
            pl.BlockSpec((D, D), lambda b: (0, 0)),
            pl.BlockSpec((D,), lambda b: (0,)),
            pl.BlockSpec((D, D), lambda b: (0, 0)),
            pl.BlockSpec((D,), lambda b: (0,)),
            pl.BlockSpec((D, D), lambda b: (0, 0)),
            pl.BlockSpec((D,), lambda b: (0,)),
        ],
        out_specs=pl.BlockSpec((1, S, D), lambda b: (b, 0, 0)),
        out_shape=jax.ShapeDtypeStruct((B, S, D), jnp.float32),
        scratch_shapes=[pltpu.VMEM((16, DH, D), jnp.float32)],
    )(shelf_embs, product_embs, supply, W_q, b_q, W_k, b_k, W_v, b_v, W_o, b_o)
    return out


# speedup vs baseline: 1.9091x; 1.0334x over previous
import jax
import jax.numpy as jnp
from jax import lax
from jax.experimental import pallas as pl
from jax.experimental.pallas import tpu as pltpu

B, S, P = 2, 128, 256
D = 1024
H = 16
DH = D // H

G = 2
HPG = H // G
DG = D // G
RG = S // G

_XWT = (((1,), (1,)), ((), ()))


def _fused_body(shelf_ref, product_ref, supply_ref, wq_ref, bq_ref,
                wk_ref, bk_ref, wv_ref, bv_ref, wo_ref, bo_ref,
                out_ref, wo3_s):
    @pl.when((pl.program_id(0) == 0) & (pl.program_id(1) == 0))
    def _prep():
        wo3_s[...] = wo_ref[...].T.reshape(16, DH, D)

    x_s = shelf_ref[0]
    x_p = product_ref[0]

    q = lax.dot_general(x_s, wq_ref[...], _XWT,
                        preferred_element_type=jnp.float32) + bq_ref[...]
    k = lax.dot_general(x_p, wk_ref[...], _XWT,
                        preferred_element_type=jnp.float32) + bk_ref[...]
    v = lax.dot_general(x_p, wv_ref[...], _XWT,
                        preferred_element_type=jnp.float32) + bv_ref[...]

    q4 = q.reshape(S, HPG, DH)
    k4 = k.reshape(P, HPG, DH)
    v4 = v.reshape(P, HPG, DH)

    logits = lax.dot_general(
        q4, k4,
        dimension_numbers=(((2,), (2,)), ((1,), (1,))),
        preferred_element_type=jnp.float32,
    ) * (1.0 / (DH ** 0.5))

    mask = (supply_ref[0] > 0)[None, :, :]
    m = jnp.max(logits, axis=2, keepdims=True)
    e = jnp.where(mask, jnp.exp(logits - m), 0.0)
    den = jnp.sum(e, axis=2, keepdims=True)
    w = e / (den + 1e-9)

    attn = lax.dot_general(
        w, v4,
        dimension_numbers=(((2,), (0,)), ((0,), (1,))),
        preferred_element_type=jnp.float32,
    )

    a4 = attn.reshape(HPG, 8, 16, DH)
    t = lax.dot_general(
        a4, wo3_s[...],
        dimension_numbers=(((3,), (1,)), ((2,), (0,))),
        preferred_element_type=jnp.float32,
    )
    out_ref[0] = jnp.sum(t, axis=0).reshape(RG, D) + bo_ref[...]


@jax.jit
def kernel(shelf_embs, product_embs, supply, W_q, b_q, W_k, b_k, W_v, b_v, W_o, b_o):
    out = pl.pallas_call(
        _fused_body,
        grid=(G, B),
        in_specs=[
            pl.BlockSpec((1, S, D), lambda g, b: (b, 0, 0)),
            pl.BlockSpec((1, P, D), lambda g, b: (b, 0, 0)),
            pl.BlockSpec((1, S, P), lambda g, b: (b, 0, 0)),
            pl.BlockSpec((DG, D), lambda g, b: (g, 0)),
            pl.BlockSpec((DG,), lambda g, b: (g,)),
            pl.BlockSpec((DG, D), lambda g, b: (g, 0)),
            pl.BlockSpec((DG,), lambda g, b: (g,)),
            pl.BlockSpec((DG, D), lambda g, b: (g, 0)),
            pl.BlockSpec((DG,), lambda g, b: (g,)),
            pl.BlockSpec((D, D), lambda g, b: (0, 0)),
            pl.BlockSpec((D,), lambda g, b: (0,)),
        ],
        out_specs=pl.BlockSpec((1, RG, D), lambda g, b: (b, g, 0)),
        out_shape=jax.ShapeDtypeStruct((B, S, D), jnp.float32),
        scratch_shapes=[pltpu.VMEM((16, DH, D), jnp.float32)],
    )(shelf_embs, product_embs, supply, W_q, b_q, W_k, b_k, W_v, b_v, W_o, b_o)
    return out
